# Initial kernel scaffold; baseline (speedup 1.0000x reference)
import jax
import jax.numpy as jnp
from jax.experimental import pallas as pl


def kernel(vertices, elements, field):
    raise NotImplementedError("write your pallas kernel here")


# speedup vs baseline: 20.8036x; 20.8036x over previous
import functools

import jax
import jax.numpy as jnp
from jax import lax
from jax.experimental import pallas as pl
from jax.experimental.pallas import tpu as pltpu
from jax.experimental.pallas import tpu_sc as plsc

N_VERT = 100000
N_ELEM = 200000
D = 16

NC = 2
NS = 16
NW = NC * NS

C = 256
K = C // 128
CPW = 25
EPW = C * CPW
M_PAD = EPW * NW

NV_PAD = N_VERT + 8
ROWS_PER_TILE = 6256
N_OUT = ROWS_PER_TILE * NS
ACC_PER_TILE = 6400
N_ACC = ACC_PER_TILE * NS


def _sc_body(v_hbm, e0_hbm, e1_hbm, e2_hbm, f_hbm, out_hbm,
             idx0, idx1, idx2, vr0, vr1, vr2, fr0, fr1, fr2,
             l00b, l01b, l02b, l11b, l12b, l22b, acc, vsem, fsem):
    cid = lax.axis_index("c")
    sid = lax.axis_index("s")
    wid = sid * NC + cid
    lane = lax.iota(jnp.int32, 16)
    zeros16 = jnp.zeros((16,), jnp.int32)
    ones16 = zeros16 + 1

    idxs = (idx0, idx1, idx2)
    vrs = (vr0, vr1, vr2)
    frs = (fr0, fr1, fr2)

    def _zero_row(i, _):
        fr0[i] = jnp.zeros((D,), jnp.float32)
        return 0
    lax.fori_loop(0, C, _zero_row, 0)
    zbase = sid * ACC_PER_TILE

    def _zero_acc(z, _):
        pltpu.sync_copy(fr0, acc.at[pl.ds(zbase + z * C, C)])
        return 0
    lax.fori_loop(0, ACC_PER_TILE // C, _zero_acc, 0)
    plsc.subcore_barrier()

    def _chunk(k, _):
        row0 = wid * (CPW * K) + k * K
        pltpu.sync_copy(e0_hbm.at[pl.ds(row0, K)], idx0)
        pltpu.sync_copy(e1_hbm.at[pl.ds(row0, K)], idx1)
        pltpu.sync_copy(e2_hbm.at[pl.ds(row0, K)], idx2)

        vdescs = []
        fdescs = []
        for c in range(3):
            for j in range(K):
                vdescs.append(pltpu.async_copy(
                    v_hbm.at[idxs[c].at[j]],
                    vrs[c].at[pl.ds(j * 128, 128)], vsem))
        for c in range(3):
            for j in range(K):
                fdescs.append(pltpu.async_copy(
                    f_hbm.at[idxs[c].at[j]],
                    frs[c].at[pl.ds(j * 128, 128)], fsem))
        for d_ in vdescs:
            d_.wait()

        def _geo(g, _):
            r = g * 16 + lane
            x0 = plsc.load_gather(vr0, [r, zeros16])
            y0 = plsc.load_gather(vr0, [r, ones16])
            x1 = plsc.load_gather(vr1, [r, zeros16])
            y1 = plsc.load_gather(vr1, [r, ones16])
            x2 = plsc.load_gather(vr2, [r, zeros16])
            y2 = plsc.load_gather(vr2, [r, ones16])
            area2 = (x1 - x0) * (y2 - y0) - (y1 - y0) * (x2 - x0)
            abs2 = jnp.abs(area2)
            inv = 1.0 / abs2
            area = 0.5 * abs2
            g0x = (y2 - y1) * inv
            g0y = (x1 - x2) * inv
            g1x = (y0 - y2) * inv
            g1y = (x2 - x0) * inv
            g2x = (y1 - y0) * inv
            g2y = (x0 - x1) * inv
            sl = pl.ds(g * 16, 16)
            l00b[sl] = area * (g0x * g0x + g0y * g0y)
            l01b[sl] = area * (g0x * g1x + g0y * g1y)
            l02b[sl] = area * (g0x * g2x + g0y * g2y)
            l11b[sl] = area * (g1x * g1x + g1y * g1y)
            l12b[sl] = area * (g1x * g2x + g1y * g2y)
            l22b[sl] = area * (g2x * g2x + g2y * g2y)
            return 0
        lax.fori_loop(0, C // 16, _geo, 0)

        for d_ in fdescs:
            d_.wait()

        def _ele(e, _):
            ev = zeros16 + e
            a00 = plsc.load_gather(l00b, [ev])
            a01 = plsc.load_gather(l01b, [ev])
            a02 = plsc.load_gather(l02b, [ev])
            a11 = plsc.load_gather(l11b, [ev])
            a12 = plsc.load_gather(l12b, [ev])
            a22 = plsc.load_gather(l22b, [ev])
            f0 = fr0[e]
            f1 = fr1[e]
            f2 = fr2[e]
            fr0[e] = a00 * f0 + a01 * f1 + a02 * f2
            fr1[e] = a01 * f0 + a11 * f1 + a12 * f2
            fr2[e] = a02 * f0 + a12 * f1 + a22 * f2
            return 0
        lax.fori_loop(0, C, _ele, 0)

        for c in range(3):
            for j in range(K):
                pltpu.sync_copy(frs[c].at[pl.ds(j * 128, 128)],
                                acc.at[idxs[c].at[j]], add=True)
        return 0

    lax.fori_loop(0, CPW, _chunk, 0)

    plsc.subcore_barrier()
    obase = sid * ROWS_PER_TILE

    def _out_cp(z, _):
        pltpu.sync_copy(acc.at[pl.ds(obase + z * C, C)], fr0)
        pltpu.sync_copy(fr0, out_hbm.at[pl.ds(cid * N_OUT + obase + z * C, C)])
        return 0
    lax.fori_loop(0, ROWS_PER_TILE // C, _out_cp, 0)
    orem = ROWS_PER_TILE % C
    if orem:
        ob2 = obase + (ROWS_PER_TILE // C) * C
        pltpu.sync_copy(acc.at[pl.ds(ob2, orem)], fr0.at[pl.ds(0, orem)])
        pltpu.sync_copy(fr0.at[pl.ds(0, orem)],
                        out_hbm.at[pl.ds(cid * N_OUT + ob2, orem)])


@functools.partial(
    pl.kernel,
    out_type=jax.ShapeDtypeStruct((2 * N_OUT, D), jnp.float32),
    mesh=plsc.VectorSubcoreMesh(core_axis_name="c", subcore_axis_name="s"),
    scratch_types=[
        pltpu.VMEM((K, 128), jnp.int32),
        pltpu.VMEM((K, 128), jnp.int32),
        pltpu.VMEM((K, 128), jnp.int32),
        pltpu.VMEM((C, 2), jnp.float32),
        pltpu.VMEM((C, 2), jnp.float32),
        pltpu.VMEM((C, 2), jnp.float32),
        pltpu.VMEM((C, D), jnp.float32),
        pltpu.VMEM((C, D), jnp.float32),
        pltpu.VMEM((C, D), jnp.float32),
        pltpu.VMEM((C,), jnp.float32),
        pltpu.VMEM((C,), jnp.float32),
        pltpu.VMEM((C,), jnp.float32),
        pltpu.VMEM((C,), jnp.float32),
        pltpu.VMEM((C,), jnp.float32),
        pltpu.VMEM((C,), jnp.float32),
        pltpu.VMEM_SHARED((N_ACC, D), jnp.float32),
        pltpu.SemaphoreType.DMA,
        pltpu.SemaphoreType.DMA,
    ],
    compiler_params=pltpu.CompilerParams(
        needs_layout_passes=False, use_tc_tiling_on_sc=False),
)
def _sc_laplacian(v_hbm, e0_hbm, e1_hbm, e2_hbm, f_hbm, out_hbm, *scratch):
    _sc_body(v_hbm, e0_hbm, e1_hbm, e2_hbm, f_hbm, out_hbm, *scratch)


def _add_body(a_ref, b_ref, o_ref):
    o_ref[...] = a_ref[...] + b_ref[...]


def kernel(vertices, elements, field):
    e = elements.astype(jnp.int32)
    pad = jnp.full((M_PAD - N_ELEM,), N_VERT, jnp.int32)
    e0 = jnp.concatenate([e[:, 0], pad]).reshape(M_PAD // 128, 128)
    e1 = jnp.concatenate([e[:, 1], pad]).reshape(M_PAD // 128, 128)
    e2 = jnp.concatenate([e[:, 2], pad]).reshape(M_PAD // 128, 128)
    v_p = jnp.zeros((NV_PAD, 2), jnp.float32).at[:N_VERT].set(vertices)
    f_p = jnp.zeros((NV_PAD, D), jnp.float32).at[:N_VERT].set(
        field.astype(jnp.float32))

    partials = _sc_laplacian(v_p, e0, e1, e2, f_p)

    a = partials[:N_VERT].reshape(N_VERT * D // 128, 128)
    b = partials[N_OUT:N_OUT + N_VERT].reshape(N_VERT * D // 128, 128)
    rows = N_VERT * D // 128
    out = pl.pallas_call(
        _add_body,
        out_shape=jax.ShapeDtypeStruct((rows, 128), jnp.float32),
    )(a, b)
    return out.reshape(N_VERT, D)


# speedup vs baseline: 23.8608x; 1.1470x over previous
import functools

import jax
import jax.numpy as jnp
from jax import lax
from jax.experimental import pallas as pl
from jax.experimental.pallas import tpu as pltpu
from jax.experimental.pallas import tpu_sc as plsc

N_VERT = 100000
N_ELEM = 200000
D = 16
NVP = N_VERT // 8

NC = 2
NS = 16
NW = NC * NS

B = 128
NB = 2
C = B * NB
NCHUNK = -(-N_ELEM // C)
TPW = -(-NCHUNK // NW)
LAST_BASE = N_ELEM - C

DUMMY = N_VERT
ROWS_PER_TILE = 6256
N_OUT = ROWS_PER_TILE * NS
ACC_PER_TILE = 6272
N_ACC = ACC_PER_TILE * NS


def _sc_body(v_hbm, e_hbm, f_hbm, out_hbm, eblk, gidx, cidx, fidx, sidx,
             vr, fr, acc, vsem, fsem):
    cid = lax.axis_index("c")
    sid = lax.axis_index("s")
    wid = sid * NC + cid
    lane = lax.iota(jnp.int32, 16)
    zeros16 = jnp.zeros((16,), jnp.int32)
    ones16 = zeros16 + 1
    twos16 = zeros16 + 2
    dm16 = jnp.full((16,), DUMMY, jnp.int32)
    zbuf = fr[0][0]

    def _zero_row(i, _):
        zbuf[i] = jnp.zeros((D,), jnp.float32)
        return 0
    lax.fori_loop(0, B, _zero_row, 0)
    zbase = sid * ACC_PER_TILE

    def _zero_acc(z, _):
        pltpu.sync_copy(zbuf, acc.at[pl.ds(zbase + z * B, B)])
        return 0
    lax.fori_loop(0, ACC_PER_TILE // B, _zero_acc, 0)
    plsc.subcore_barrier()

    def _chunk(t, _):
        chunk = t * NW + wid
        cbase = chunk * C
        base = jnp.minimum(cbase, LAST_BASE)
        shift = cbase - base

        pltpu.sync_copy(e_hbm.at[pl.ds(base, C)], eblk)

        for j in range(NB):
            gj, cj, fj, sj = gidx[j], cidx[j], fidx[j], sidx[j]

            def _bidx(g, _):
                r = j * B + g * 16 + lane
                valid = r >= shift
                i0 = plsc.load_gather(eblk, [r, zeros16])
                i1 = plsc.load_gather(eblk, [r, ones16])
                i2 = plsc.load_gather(eblk, [r, twos16])
                sl = pl.ds(g * 16, 16)
                gj[0][sl] = i0 >> 3
                gj[1][sl] = i1 >> 3
                gj[2][sl] = i2 >> 3
                cj[0][sl] = (i0 & 7) << 1
                cj[1][sl] = (i1 & 7) << 1
                cj[2][sl] = (i2 & 7) << 1
                fj[0][sl] = i0
                fj[1][sl] = i1
                fj[2][sl] = i2
                sj[0][sl] = jnp.where(valid, i0, dm16)
                sj[1][sl] = jnp.where(valid, i1, dm16)
                sj[2][sl] = jnp.where(valid, i2, dm16)
                return 0
            lax.fori_loop(0, B // 16, _bidx, 0)

        descs = []
        for j in range(NB):
            for c in range(3):
                descs.append(pltpu.async_copy(
                    v_hbm.at[gidx[j][c]], vr[j][c], vsem))
        for j in range(NB):
            for c in range(3):
                descs.append(pltpu.async_copy(
                    f_hbm.at[fidx[j][c]], fr[j][c], fsem))
        for d_ in descs:
            d_.wait()

        for j in range(NB):
            v0, v1, v2 = vr[j]
            f0b, f1b, f2b = fr[j]
            c0b, c1b, c2b = cidx[j]

            def _grp(g, _):
                r = g * 16 + lane
                sl = pl.ds(g * 16, 16)
                c0 = c0b[sl]
                c1 = c1b[sl]
                c2 = c2b[sl]
                x0 = plsc.load_gather(v0, [r, c0])
                y0 = plsc.load_gather(v0, [r, c0 + 1])
                x1 = plsc.load_gather(v1, [r, c1])
                y1 = plsc.load_gather(v1, [r, c1 + 1])
                x2 = plsc.load_gather(v2, [r, c2])
                y2 = plsc.load_gather(v2, [r, c2 + 1])
                area2 = (x1 - x0) * (y2 - y0) - (y1 - y0) * (x2 - x0)
                abs2 = jnp.abs(area2)
                inv = 1.0 / abs2
                area = 0.5 * abs2
                g0x = (y2 - y1) * inv
                g0y = (x1 - x2) * inv
                g1x = (y0 - y2) * inv
                g1y = (x2 - x0) * inv
                g2x = (y1 - y0) * inv
                g2y = (x0 - x1) * inv
                a00 = area * (g0x * g0x + g0y * g0y)
                a01 = area * (g0x * g1x + g0y * g1y)
                a02 = area * (g0x * g2x + g0y * g2y)
                a11 = area * (g1x * g1x + g1y * g1y)
                a12 = area * (g1x * g2x + g1y * g2y)
                a22 = area * (g2x * g2x + g2y * g2y)
                for i in range(16):
                    iv = jnp.full((16,), i, jnp.int32)

                    def bc(a):
                        return jnp.take_along_axis(
                            a, iv, axis=0, mode="promise_in_bounds")
                    b00, b01, b02 = bc(a00), bc(a01), bc(a02)
                    b11, b12, b22 = bc(a11), bc(a12), bc(a22)
                    e = g * 16 + i
                    f0 = f0b[e]
                    f1 = f1b[e]
                    f2 = f2b[e]
                    f0b[e] = b00 * f0 + b01 * f1 + b02 * f2
                    f1b[e] = b01 * f0 + b11 * f1 + b12 * f2
                    f2b[e] = b02 * f0 + b12 * f1 + b22 * f2
                return 0
            lax.fori_loop(0, B // 16, _grp, 0)

        for j in range(NB):
            for c in range(3):
                pltpu.sync_copy(fr[j][c], acc.at[sidx[j][c]], add=True)
        return 0

    lax.fori_loop(0, TPW, _chunk, 0)

    plsc.subcore_barrier()
    obase = sid * ROWS_PER_TILE

    def _out_cp(z, _):
        pltpu.sync_copy(acc.at[pl.ds(obase + z * B, B)], zbuf)
        pltpu.sync_copy(zbuf, out_hbm.at[cid, pl.ds(obase + z * B, B)])
        return 0
    lax.fori_loop(0, ROWS_PER_TILE // B, _out_cp, 0)
    orem = ROWS_PER_TILE % B
    if orem:
        ob2 = obase + (ROWS_PER_TILE // B) * B
        pltpu.sync_copy(acc.at[pl.ds(ob2, orem)], zbuf.at[pl.ds(0, orem)])
        pltpu.sync_copy(zbuf.at[pl.ds(0, orem)],
                        out_hbm.at[cid, pl.ds(ob2, orem)])


@functools.partial(
    pl.kernel,
    out_type=jax.ShapeDtypeStruct((2, N_OUT, D), jnp.float32),
    mesh=plsc.VectorSubcoreMesh(core_axis_name="c", subcore_axis_name="s"),
    scratch_types=[
        pltpu.VMEM((C, 3), jnp.int32),
        [[pltpu.VMEM((B,), jnp.int32) for _ in range(3)]
         for _ in range(NB)],
        [[pltpu.VMEM((B,), jnp.int32) for _ in range(3)]
         for _ in range(NB)],
        [[pltpu.VMEM((B,), jnp.int32) for _ in range(3)]
         for _ in range(NB)],
        [[pltpu.VMEM((B,), jnp.int32) for _ in range(3)]
         for _ in range(NB)],
        [[pltpu.VMEM((B, D), jnp.float32) for _ in range(3)]
         for _ in range(NB)],
        [[pltpu.VMEM((B, D), jnp.float32) for _ in range(3)]
         for _ in range(NB)],
        pltpu.VMEM_SHARED((N_ACC, D), jnp.float32),
        pltpu.SemaphoreType.DMA,
        pltpu.SemaphoreType.DMA,
    ],
    compiler_params=pltpu.CompilerParams(
        needs_layout_passes=False, use_tc_tiling_on_sc=False),
)
def _sc_laplacian(v_hbm, e_hbm, f_hbm, out_hbm, *scratch):
    _sc_body(v_hbm, e_hbm, f_hbm, out_hbm, *scratch)


def _add_body(a_ref, b_ref, o_ref):
    o_ref[...] = a_ref[0] + b_ref[0]


_BLK = 4000


def kernel(vertices, elements, field):
    e = elements.astype(jnp.int32)
    v_packed = vertices.reshape(NVP, 16)
    partials = _sc_laplacian(v_packed, e, field)

    out = pl.pallas_call(
        _add_body,
        grid=(N_VERT // _BLK,),
        in_specs=[pl.BlockSpec((1, _BLK, D), lambda i: (0, i, 0)),
                  pl.BlockSpec((1, _BLK, D), lambda i: (1, i, 0))],
        out_specs=pl.BlockSpec((_BLK, D), lambda i: (i, 0)),
        out_shape=jax.ShapeDtypeStruct((N_VERT, D), jnp.float32),
    )(partials, partials)
    return out
